# Initial kernel scaffold; baseline (speedup 1.0000x reference)
import jax
import jax.numpy as jnp
from jax.experimental import pallas as pl


def kernel(indices, table):
    raise NotImplementedError("write your pallas kernel here")


# speedup vs baseline: 1.4586x; 1.4586x over previous
import functools

import jax
import jax.numpy as jnp
from jax import lax
from jax.experimental import pallas as pl
from jax.experimental.pallas import tpu as pltpu
from jax.experimental.pallas import tpu_sc as plsc


def _build(B, D, chunk):
    info = plsc.get_sparse_core_info()
    NC, NS = info.num_cores, info.num_subcores
    NW = NC * NS
    assert B % NW == 0
    b_per_w = B // NW
    assert b_per_w % chunk == 0
    n_chunks = b_per_w // chunk
    mesh = plsc.VectorSubcoreMesh(core_axis_name="c", subcore_axis_name="s")

    @functools.partial(
        pl.kernel,
        mesh=mesh,
        out_type=jax.ShapeDtypeStruct((B, D), jnp.float32),
        compiler_params=pltpu.CompilerParams(use_tc_tiling_on_sc=False),
        scratch_types=[
            pltpu.VMEM((chunk,), jnp.int32),
            pltpu.VMEM((chunk, D), jnp.float32),
            pltpu.SemaphoreType.DMA,
        ],
    )
    def gather_kernel(table_hbm, idx_hbm, out_hbm, idx_v, rows_v, sem):
        wid = lax.axis_index("s") * NC + lax.axis_index("c")
        base = wid * b_per_w

        def body(i, carry):
            off = pl.multiple_of(base + i * chunk, chunk)
            pltpu.sync_copy(idx_hbm.at[pl.ds(off, chunk)], idx_v)
            pltpu.async_copy(table_hbm.at[idx_v], rows_v, sem).wait()
            pltpu.sync_copy(rows_v, out_hbm.at[pl.ds(off, chunk)])
            return carry

        lax.fori_loop(0, n_chunks, body, 0)

    return gather_kernel


@jax.jit
def kernel(indices, table):
    B = indices.shape[0] * indices.shape[1]
    D = table.shape[1]
    idx_flat = indices.reshape(-1).astype(jnp.int32)
    out = _build(B, D, 1024)(table, idx_flat)
    return out.reshape(indices.shape + (D,))


# speedup vs baseline: 1.4927x; 1.0233x over previous
import functools

import jax
import jax.numpy as jnp
from jax import lax
from jax.experimental import pallas as pl
from jax.experimental.pallas import tpu as pltpu
from jax.experimental.pallas import tpu_sc as plsc

_CHUNK = 640
_NBUF = 4


def _build(B, D, chunk, nbuf):
    info = plsc.get_sparse_core_info()
    NC, NS = info.num_cores, info.num_subcores
    NW = NC * NS
    assert B % NW == 0
    b_per_w = B // NW
    assert b_per_w % chunk == 0
    n_chunks = b_per_w // chunk
    assert n_chunks % nbuf == 0
    n_groups = n_chunks // nbuf
    mesh = plsc.VectorSubcoreMesh(core_axis_name="c", subcore_axis_name="s")

    @functools.partial(
        pl.kernel,
        mesh=mesh,
        out_type=jax.ShapeDtypeStruct((B, D), jnp.float32),
        compiler_params=pltpu.CompilerParams(use_tc_tiling_on_sc=False),
        scratch_types=(
            [pltpu.VMEM((b_per_w,), jnp.int32),
             pltpu.VMEM((nbuf, chunk, D), jnp.float32)]
            + [pltpu.SemaphoreType.DMA] * (2 * nbuf)
        ),
    )
    def gather_kernel(table_hbm, idx_hbm, out_hbm, idx_all, rows_v, *sems):
        gsems, osems = sems[:nbuf], sems[nbuf:]
        wid = lax.axis_index("s") * NC + lax.axis_index("c")
        base = pl.multiple_of(wid * b_per_w, 8)

        pltpu.sync_copy(idx_hbm.at[pl.ds(base, b_per_w)], idx_all)

        def gather_start(i, b):
            loff = pl.multiple_of(i * chunk, 8)
            pltpu.async_copy(
                table_hbm.at[idx_all.at[pl.ds(loff, chunk)]],
                rows_v.at[b], gsems[b])

        def gather_wait(b):
            pltpu.make_async_copy(
                table_hbm.at[idx_all.at[pl.ds(0, chunk)]],
                rows_v.at[b], gsems[b]).wait()

        def out_start(i, b):
            off = pl.multiple_of(base + i * chunk, 8)
            pltpu.async_copy(rows_v.at[b], out_hbm.at[pl.ds(off, chunk)],
                             osems[b])

        def out_wait(b):
            pltpu.make_async_copy(rows_v.at[b], out_hbm.at[pl.ds(0, chunk)],
                                  osems[b]).wait()

        for b in range(nbuf):
            gather_start(b, b)

        def group(gi, carry):
            i0 = gi * nbuf
            for b in range(nbuf):
                gather_wait(b)
                out_start(i0 - nbuf + b, b)
            for b in range(nbuf):
                out_wait(b)
                gather_start(i0 + b, b)
            return carry

        lax.fori_loop(1, n_groups, group, 0, unroll=False)

        i0 = (n_groups - 1) * nbuf
        for b in range(nbuf):
            gather_wait(b)
            out_start(i0 + b, b)
        for b in range(nbuf):
            out_wait(b)

    return gather_kernel


@jax.jit
def kernel(indices, table):
    B = indices.shape[0] * indices.shape[1]
    D = table.shape[1]
    idx_flat = indices.reshape(-1).astype(jnp.int32)
    out = _build(B, D, _CHUNK, _NBUF)(table, idx_flat)
    return out.reshape(indices.shape + (D,))
